# Initial kernel scaffold; baseline (speedup 1.0000x reference)
import jax
import jax.numpy as jnp
from jax.experimental import pallas as pl


def kernel(x, LRF_getter):
    raise NotImplementedError("write your pallas kernel here")


# speedup vs baseline: 62.1966x; 62.1966x over previous
import functools

import jax
import jax.numpy as jnp
from jax import lax
from jax.experimental import pallas as pl
from jax.experimental.pallas import tpu as pltpu
from jax.experimental.pallas import tpu_sc as plsc

B, N, P = 16, 2048, 256
M, L = 512, 9

NC = 2
NS = 16
LANES = 16
NW = NC * NS

ROWS = M * P
RPW = ROWS // NW
CH = 256
NCHUNK = RPW // CH
GIDX = 128
GB = CH * L // GIDX
IDX_BLOCKS = ROWS * L // GIDX


def _sc_gather_max(xt, idx_blocks):
    mesh = plsc.VectorSubcoreMesh(core_axis_name="c", subcore_axis_name="s")

    @functools.partial(
        pl.kernel,
        mesh=mesh,
        compiler_params=pltpu.CompilerParams(use_tc_tiling_on_sc=False),
        out_type=jax.ShapeDtypeStruct((ROWS, LANES), jnp.float32),
        scratch_types=[
            pltpu.VMEM((RPW * L // GIDX, GIDX), jnp.int32),
            pltpu.VMEM((CH * L, LANES), jnp.float32),
            pltpu.VMEM((CH, LANES), jnp.float32),
            pltpu.SemaphoreType.DMA,
        ],
    )
    def k(xt_hbm, idx_hbm, out_hbm, idx_v, rows_v, out_v, sem):
        wid = lax.axis_index("s") * NC + lax.axis_index("c")
        blk_per_w = RPW * L // GIDX
        pltpu.sync_copy(idx_hbm.at[pl.ds(wid * blk_per_w, blk_per_w), :], idx_v)

        def chunk_body(c, carry):
            base_row = wid * RPW + c * CH
            copies = []
            for j in range(GB):
                copies.append(pltpu.async_copy(
                    xt_hbm.at[idx_v.at[c * GB + j]],
                    rows_v.at[pl.ds(j * GIDX, GIDX), :],
                    sem,
                ))
            for cp in copies:
                cp.wait()

            def row_body(r, carry2):
                v = rows_v[r * L]
                for l in range(1, L):
                    v = jnp.maximum(v, rows_v[r * L + l])
                out_v[r] = v
                return carry2

            lax.fori_loop(0, CH, row_body, 0)
            pltpu.sync_copy(out_v, out_hbm.at[pl.ds(base_row, CH), :])
            return carry

        lax.fori_loop(0, NCHUNK, chunk_body, 0)

    return k(xt, idx_blocks)


def kernel(x, LRF_getter):
    xt = jnp.transpose(x, (1, 2, 0)).reshape(N * P, B)
    idx_n = LRF_getter[..., 0]
    idx_c = LRF_getter[..., 1]
    flat = (idx_n * P + idx_c).reshape(IDX_BLOCKS, GIDX)
    out_t = _sc_gather_max(xt, flat)
    return jnp.transpose(out_t.reshape(M, P, B), (2, 0, 1))


# speedup vs baseline: 67.6224x; 1.0872x over previous
import functools

import jax
import jax.numpy as jnp
from jax import lax
from jax.experimental import pallas as pl
from jax.experimental.pallas import tpu as pltpu
from jax.experimental.pallas import tpu_sc as plsc

B, N, P = 16, 2048, 256
M, L = 512, 9

NC = 2
NS = 16
LANES = 16
NW = NC * NS

ROWS = M * P
RPW = ROWS // NW
CH = 256
NCHUNK = RPW // CH
GIDX = 128
GB = CH * L // GIDX
IDX_BLOCKS = ROWS * L // GIDX


def _sc_gather_max(xt, idx_blocks):
    mesh = plsc.VectorSubcoreMesh(core_axis_name="c", subcore_axis_name="s")

    @functools.partial(
        pl.kernel,
        mesh=mesh,
        compiler_params=pltpu.CompilerParams(use_tc_tiling_on_sc=False),
        out_type=jax.ShapeDtypeStruct((ROWS, LANES), jnp.float32),
        scratch_types=[
            pltpu.VMEM((RPW * L // GIDX, GIDX), jnp.int32),
            pltpu.VMEM((2 * CH * L, LANES), jnp.float32),
            pltpu.VMEM((2 * CH, LANES), jnp.float32),
            pltpu.SemaphoreType.DMA,
            pltpu.SemaphoreType.DMA,
            pltpu.SemaphoreType.DMA,
            pltpu.SemaphoreType.DMA,
        ],
    )
    def k(xt_hbm, idx_hbm, out_hbm, idx_v, rows_v, out_v,
          gsem0, gsem1, osem0, osem1):
        wid = lax.axis_index("s") * NC + lax.axis_index("c")
        gsems = (gsem0, gsem1)
        osems = (osem0, osem1)
        blk_per_w = RPW * L // GIDX
        pltpu.sync_copy(idx_hbm.at[pl.ds(wid * blk_per_w, blk_per_w), :], idx_v)

        def fire(c):
            par = c % 2
            for j in range(GB):
                pltpu.async_copy(
                    xt_hbm.at[idx_v.at[c * GB + j]],
                    rows_v.at[pl.ds(par * CH * L + j * GIDX, GIDX), :],
                    gsems[par],
                )

        def drain(c):
            par = c % 2
            for j in range(GB):
                pltpu.make_async_copy(
                    xt_hbm.at[idx_v.at[c * GB + j]],
                    rows_v.at[pl.ds(par * CH * L + j * GIDX, GIDX), :],
                    gsems[par],
                ).wait()

        fire(0)
        for c in range(NCHUNK):
            par = c % 2
            if c + 1 < NCHUNK:
                fire(c + 1)
            if c >= 2:
                pltpu.make_async_copy(
                    out_v.at[pl.ds(par * CH, CH), :],
                    out_hbm.at[pl.ds(wid * RPW + (c - 2) * CH, CH), :],
                    osems[par],
                ).wait()
            drain(c)

            def row_body(r, carry2, _par=par):
                base = _par * CH * L + r * L
                v = rows_v[base]
                for l in range(1, L):
                    v = jnp.maximum(v, rows_v[base + l])
                out_v[_par * CH + r] = v
                return carry2

            lax.fori_loop(0, CH, row_body, 0, unroll=2)
            pltpu.async_copy(
                out_v.at[pl.ds(par * CH, CH), :],
                out_hbm.at[pl.ds(wid * RPW + c * CH, CH), :],
                osems[par],
            )
        for c in (NCHUNK - 2, NCHUNK - 1):
            par = c % 2
            pltpu.make_async_copy(
                out_v.at[pl.ds(par * CH, CH), :],
                out_hbm.at[pl.ds(wid * RPW + c * CH, CH), :],
                osems[par],
            ).wait()

    return k(xt, idx_blocks)


def kernel(x, LRF_getter):
    xt = jnp.transpose(x, (1, 2, 0)).reshape(N * P, B)
    idx_n = LRF_getter[..., 0]
    idx_c = LRF_getter[..., 1]
    flat = (idx_n * P + idx_c).reshape(IDX_BLOCKS, GIDX)
    out_t = _sc_gather_max(xt, flat)
    return jnp.transpose(out_t.reshape(M, P, B), (2, 0, 1))
